# Initial kernel scaffold; baseline (speedup 1.0000x reference)
import jax
import jax.numpy as jnp
from jax.experimental import pallas as pl


def kernel(x, edge_index, edge_weight, W, b):
    raise NotImplementedError("write your pallas kernel here")


# speedup vs baseline: 4.3129x; 4.3129x over previous
import functools

import jax
import jax.numpy as jnp
from jax import lax
from jax.experimental import pallas as pl
from jax.experimental.pallas import tpu as pltpu
from jax.experimental.pallas import tpu_sc as plsc

NC = 2
NS = 16
L = 16

CHUNK = 80
GROUPS = CHUNK // L


def _mesh():
  return plsc.VectorSubcoreMesh(
      core_axis_name="c", subcore_axis_name="s", num_cores=NC,
      num_subcores=NS)


def _make_deg_kernel(n_pad, e):
  ept = e // NS
  nch = ept // CHUNK
  spt = n_pad // NS

  @functools.partial(
      pl.kernel,
      out_type=jax.ShapeDtypeStruct((n_pad,), jnp.float32),
      mesh=_mesh(),
      scratch_types=[
          pltpu.VMEM((ept,), jnp.int32),
          pltpu.VMEM((ept,), jnp.float32),
          pltpu.VMEM((CHUNK,), jnp.int32),
          pltpu.VMEM((spt,), jnp.float32),
          pltpu.VMEM_SHARED((n_pad,), jnp.float32),
      ],
  )
  def deg_kernel(dst_hbm, ew_hbm, deg_hbm, dstb, ewb, dstv, degv, deg_sp):
    cid = lax.axis_index("c")
    sid = lax.axis_index("s")

    @pl.when(cid == 0)
    def _():
      def zb(i, carry):
        degv[pl.ds(i * L, L)] = jnp.zeros((L,), jnp.float32)
        return carry
      lax.fori_loop(0, spt // L, zb, None)
      pltpu.sync_copy(degv, deg_sp.at[pl.ds(sid * spt, spt)])
      plsc.subcore_barrier()

      pltpu.sync_copy(dst_hbm.at[pl.ds(sid * ept, ept)], dstb)
      pltpu.sync_copy(ew_hbm.at[pl.ds(sid * ept, ept)], ewb)

      def cb(ch, carry):
        off = ch * CHUNK
        for g in range(GROUPS):
          dstv[pl.ds(g * L, L)] = dstb[pl.ds(off + g * L, L)]
        pltpu.sync_copy(ewb.at[pl.ds(off, CHUNK)], deg_sp.at[dstv], add=True)
        return carry
      lax.fori_loop(0, nch, cb, None)
      plsc.subcore_barrier()

      pltpu.sync_copy(deg_sp.at[pl.ds(sid * spt, spt)], degv)
      pltpu.sync_copy(degv, deg_hbm.at[pl.ds(sid * spt, spt)])

  return deg_kernel


def _make_agg_kernel(n, n_pad, d, e):
  nw = NC * NS
  epw = e // nw
  sup = 2000
  nsup = epw // sup
  nch = sup // CHUNK
  rpt = n_pad // NS

  out_sds = jax.ShapeDtypeStruct((n_pad, d), jnp.float32)

  @functools.partial(
      pl.kernel,
      out_type=(out_sds, out_sds),
      mesh=_mesh(),
      scratch_types=[
          pltpu.VMEM((sup,), jnp.int32),
          pltpu.VMEM((sup,), jnp.int32),
          pltpu.VMEM((sup,), jnp.float32),
          pltpu.VMEM((n,), jnp.float32),
          pltpu.VMEM((CHUNK,), jnp.int32),
          pltpu.VMEM((CHUNK,), jnp.int32),
          pltpu.VMEM((CHUNK, d), jnp.float32),
          pltpu.VMEM_SHARED((n_pad, d), jnp.float32),
          pltpu.SemaphoreType.DMA,
      ],
      compiler_params=pltpu.CompilerParams(needs_layout_passes=False),
  )
  def agg_kernel(src_hbm, dst_hbm, ew_hbm, dis_hbm, h_hbm, out0, out1,
                 srcb, dstb, ewb, disv, srcv, dstv, rows, acc, sem):
    cid = lax.axis_index("c")
    sid = lax.axis_index("s")
    wid = cid * NS + sid

    pltpu.sync_copy(dis_hbm.at[pl.ds(0, n)], disv)

    def zb(i, carry):
      for j in range(d // L):
        rows[i, pl.ds(j * L, L)] = jnp.zeros((L,), jnp.float32)
      return carry
    lax.fori_loop(0, CHUNK, zb, None)
    assert rpt % CHUNK == 0
    for i in range(rpt // CHUNK):
      pltpu.sync_copy(rows, acc.at[pl.ds(sid * rpt + i * CHUNK, CHUNK)])
    plsc.subcore_barrier()

    def super_body(sp, carry):
      base = wid * epw + sp * sup
      pltpu.sync_copy(src_hbm.at[pl.ds(base, sup)], srcb)
      pltpu.sync_copy(dst_hbm.at[pl.ds(base, sup)], dstb)
      pltpu.sync_copy(ew_hbm.at[pl.ds(base, sup)], ewb)

      def chunk_body(ch, carry1):
        off = ch * CHUNK
        for g in range(GROUPS):
          srcv[pl.ds(g * L, L)] = srcb[pl.ds(off + g * L, L)]
          dstv[pl.ds(g * L, L)] = dstb[pl.ds(off + g * L, L)]
        pltpu.async_copy(h_hbm.at[srcv], rows, sem).wait()
        def gbody(g, carry2):
          b = g * L
          s16 = srcv[pl.ds(b, L)]
          d16 = dstv[pl.ds(b, L)]
          ew16 = ewb[pl.ds(off + b, L)]
          c16 = plsc.load_gather(disv, [s16]) * ew16 * plsc.load_gather(
              disv, [d16])
          ridx = b + lax.iota(jnp.int32, L)
          for f in range(d):
            cidx = jnp.full((L,), f, dtype=jnp.int32)
            v = plsc.load_gather(rows, [ridx, cidx])
            plsc.store_scatter(rows, [ridx, cidx], v * c16)
          return carry2
        lax.fori_loop(0, GROUPS, gbody, None)
        pltpu.sync_copy(rows, acc.at[dstv], add=True)
        return carry1
      lax.fori_loop(0, nch, chunk_body, None)
      return carry
    lax.fori_loop(0, nsup, super_body, None)
    plsc.subcore_barrier()

    @pl.when(cid == 0)
    def _():
      pltpu.sync_copy(acc.at[pl.ds(sid * rpt, rpt)],
                      out0.at[pl.ds(sid * rpt, rpt)])

    @pl.when(cid == 1)
    def _():
      pltpu.sync_copy(acc.at[pl.ds(sid * rpt, rpt)],
                      out1.at[pl.ds(sid * rpt, rpt)])

  return agg_kernel


def _matmul_body(x_ref, w_ref, deg_ref, h_ref, dis_ref):
  h_ref[...] = jnp.dot(x_ref[...], w_ref[...],
                       preferred_element_type=jnp.float32)
  t = deg_ref[...] + 1.0
  dis_ref[...] = jnp.where(t > 0.0, lax.rsqrt(t), 0.0)


def _combine_body(p0_ref, p1_ref, h_ref, dis_ref, b_ref, o_ref):
  n = o_ref.shape[0]
  dis = dis_ref[...]
  o_ref[...] = (p0_ref[pl.ds(0, n), :] + p1_ref[pl.ds(0, n), :]
                + h_ref[...] * (dis * dis) + b_ref[...])


def kernel(x, edge_index, edge_weight, W, b):
  n, d = x.shape
  e = edge_index.shape[1]
  n_pad = ((n + (NS * L) - 1) // (NS * L)) * (NS * L)
  assert e % (NC * NS * CHUNK) == 0 and n % NS == 0 and d % L == 0

  src = edge_index[0]
  dst = edge_index[1]

  deg = _make_deg_kernel(n_pad, e)(dst, edge_weight)
  h, dis2d = pl.pallas_call(
      _matmul_body,
      out_shape=(jax.ShapeDtypeStruct((n, d), jnp.float32),
                 jax.ShapeDtypeStruct((n_pad // d, d), jnp.float32)),
  )(x, W, deg.reshape(n_pad // d, d))
  dis_full = dis2d.reshape(n_pad)
  p0, p1 = _make_agg_kernel(n, n_pad, d, e)(src, dst, edge_weight, dis_full, h)
  dis_col = dis_full[:n].reshape(n, 1)
  out = pl.pallas_call(
      _combine_body,
      out_shape=jax.ShapeDtypeStruct((n, d), jnp.float32),
  )(p0, p1, h, dis_col, b.reshape(1, d))
  return out
